# baseline (speedup 1.0000x reference)
import functools

import jax
import jax.numpy as jnp
from jax import lax
from jax.experimental import pallas as pl
from jax.experimental.pallas import tpu as pltpu
from jax.experimental.pallas import tpu_sc as plsc

NC = 2
NS = 16
LANES = 16

EMBED_DIM = 128
CHUNK = 100
NBUF = 4


def _embed_kernel(n_chunks_per_w, x_hbm, tok_hbm, pos_hbm, out_hbm,
                  idx_v, pos_v, buf, gsem, ssem):
    wid = lax.axis_index("s") * NC + lax.axis_index("c")
    row0 = wid * n_chunks_per_w

    pltpu.sync_copy(x_hbm.at[pl.ds(row0, n_chunks_per_w)], idx_v)
    pltpu.sync_copy(pos_hbm, pos_v)

    n_sub = EMBED_DIM // LANES

    def fire_gather(c, b):
        pltpu.async_copy(tok_hbm.at[idx_v.at[c]], buf.at[b], gsem[b])

    def wait_gather(c, b):
        pltpu.make_async_copy(tok_hbm.at[idx_v.at[c]],
                              buf.at[b], gsem[b]).wait()

    def fire_store(c, b):
        pltpu.async_copy(buf.at[b],
                         out_hbm.at[pl.ds((row0 + c) * CHUNK, CHUNK)],
                         ssem[b])

    def wait_store(c, b):
        pltpu.make_async_copy(buf.at[b],
                              out_hbm.at[pl.ds((row0 + c) * CHUNK, CHUNK)],
                              ssem[b]).wait()

    for b in range(NBUF - 1):
        fire_gather(b, b)

    def step(g, carry):
        for b in range(NBUF):
            c = g * NBUF + b
            wait_gather(c, b)
            po = lax.rem(c, 2) * CHUNK

            def add_body(r, carry2):
                for d in range(n_sub):
                    sl = pl.ds(d * LANES, LANES)
                    buf[b, r, sl] = buf[b, r, sl] + pos_v[po + r, sl]
                return carry2

            lax.fori_loop(0, CHUNK, add_body, 0, unroll=2)
            fire_store(c, b)

            cn = c + NBUF - 1
            bn = (b + NBUF - 1) % NBUF

            @pl.when(cn < n_chunks_per_w)
            def _():
                @pl.when(cn >= NBUF)
                def _():
                    wait_store(cn - NBUF, bn)
                fire_gather(cn, bn)
        return carry

    lax.fori_loop(0, n_chunks_per_w // NBUF, step, 0)

    for b in range(NBUF):
        c = n_chunks_per_w - NBUF + b
        wait_store(c, b)


def kernel(x, token_table, pos_table):
    B, S = x.shape
    D = token_table.shape[1]
    n_lookups = B * S
    n_w = NC * NS
    n_chunks = n_lookups // CHUNK
    n_chunks_per_w = n_chunks // n_w

    x_rows = x.reshape(n_chunks, CHUNK).astype(jnp.int32)

    mesh = plsc.VectorSubcoreMesh(
        core_axis_name="c", subcore_axis_name="s",
        num_cores=NC, num_subcores=NS)

    out_flat = pl.kernel(
        functools.partial(_embed_kernel, n_chunks_per_w),
        out_type=jax.ShapeDtypeStruct((n_lookups, D), jnp.float32),
        mesh=mesh,
        scratch_types=[
            pltpu.VMEM((n_chunks_per_w, CHUNK), jnp.int32),
            pltpu.VMEM((S, D), jnp.float32),
            pltpu.VMEM((NBUF, CHUNK, D), jnp.float32),
            [pltpu.SemaphoreType.DMA] * NBUF,
            [pltpu.SemaphoreType.DMA] * NBUF,
        ],
        compiler_params=pltpu.CompilerParams(use_tc_tiling_on_sc=False),
    )(x_rows, token_table, pos_table)

    return out_flat.reshape(B, S, D)
